# Initial kernel scaffold; baseline (speedup 1.0000x reference)
import jax
import jax.numpy as jnp
from jax.experimental import pallas as pl


def kernel(u2i_edge_index, i2u_edge_index, h_user, h_item, user_w_q, user_w_k, user_w_v, item_w_q, item_w_k, item_w_v):
    raise NotImplementedError("write your pallas kernel here")


# speedup vs baseline: 13.1215x; 13.1215x over previous
import functools

import jax
import jax.numpy as jnp
from jax import lax
from jax.experimental import pallas as pl
from jax.experimental.pallas import tpu as pltpu
from jax.experimental.pallas import tpu_sc as plsc

N = 25000
D = 128
H = 4
DH = 32
E = 400000

NC = 2
NS = 16
LN = 16
CH = 128
N_PAD = 25088
E_PAD = NS * CH * 196
EPT = E_PAD // NS
NCHUNK = EPT // CH
NR = 2
HALF = N_PAD // NR
HROWS = HALF + 128
ROWS = HROWS // NS
W = 128

BLK = 512
EB = 1024
FBLK = 784


def _proj_body(h_ref, wq_ref, wkv_ref, q_ref, kv2_ref):
    hb = h_ref[...]
    q_ref[...] = jnp.dot(hb, wq_ref[...], preferred_element_type=jnp.float32)
    for cc in range(NC):
        kv2_ref[cc] = jnp.dot(hb, wkv_ref[cc],
                              preferred_element_type=jnp.float32)


def _proj(h_pad, wq, wkv2):
    q, kv2 = pl.pallas_call(
        _proj_body,
        grid=(N_PAD // BLK,),
        in_specs=[
            pl.BlockSpec((BLK, D), lambda i: (i, 0)),
            pl.BlockSpec((D, H * DH), lambda i: (0, 0)),
            pl.BlockSpec((NC, D, 4 * DH), lambda i: (0, 0, 0)),
        ],
        out_specs=[
            pl.BlockSpec((BLK, H * DH), lambda i: (i, 0)),
            pl.BlockSpec((NC, BLK, 4 * DH), lambda i: (0, i, 0)),
        ],
        out_shape=[
            jax.ShapeDtypeStruct((N_PAD, H * DH), jnp.float32),
            jax.ShapeDtypeStruct((NC, N_PAD, 4 * DH), jnp.float32),
        ],
    )(h_pad, wq, wkv2)
    return q, kv2.reshape(NC * N_PAD, 4 * DH)


def _gather_body(dst_hbm, src2_hbm, q_hbm, kv2_hbm, qd_out, kv_out,
                 idx_v, qd, kv, sem_q, sem_k):
    c = lax.axis_index("c")
    s = lax.axis_index("s")

    def chunk_body(t, carry):
        base = s * EPT + t * CH
        pltpu.sync_copy(dst_hbm.at[pl.ds(base, CH)], idx_v)
        cq = pltpu.async_copy(q_hbm.at[idx_v], qd, sem_q)
        cq.wait()
        pltpu.sync_copy(qd, qd_out.at[c, pl.ds(base, CH)])
        pltpu.sync_copy(src2_hbm.at[c, pl.ds(base, CH)], idx_v)
        ck = pltpu.async_copy(kv2_hbm.at[idx_v], kv, sem_k)
        ck.wait()
        pltpu.sync_copy(kv, kv_out.at[c, pl.ds(base, CH)])
        return carry

    lax.fori_loop(0, NCHUNK, chunk_body, 0)


_gather = pl.kernel(
    _gather_body,
    out_type=[
        jax.ShapeDtypeStruct((NC, E_PAD, D), jnp.float32),
        jax.ShapeDtypeStruct((NC, E_PAD, D), jnp.float32),
    ],
    mesh=plsc.VectorSubcoreMesh(core_axis_name="c", subcore_axis_name="s",
                                num_cores=NC, num_subcores=NS),
    scratch_types=[
        pltpu.VMEM((CH,), jnp.int32),
        pltpu.VMEM((CH, D), jnp.float32),
        pltpu.VMEM((CH, D), jnp.float32),
        pltpu.SemaphoreType.DMA,
        pltpu.SemaphoreType.DMA,
    ],
)


def _scale_body(qd_ref, kv_ref, sv_ref):
    for c in range(NC):
        qh = qd_ref[c][:, c * 64:(c + 1) * 64]
        kvr = kv_ref[c]
        pa = jnp.exp(jnp.sum(qh[:, :DH] * kvr[:, :DH], axis=1,
                             keepdims=True))
        pb = jnp.exp(jnp.sum(qh[:, DH:2 * DH] * kvr[:, DH:2 * DH], axis=1,
                             keepdims=True))
        sv_ref[c] = jnp.concatenate([
            pa * kvr[:, 64:96],
            pb * kvr[:, 96:128],
            pa, pb,
            jnp.zeros((EB, W - 66), jnp.float32)], axis=1)


def _scale(qd_all, kv_all):
    return pl.pallas_call(
        _scale_body,
        grid=(E_PAD // EB,),
        in_specs=[
            pl.BlockSpec((NC, EB, D), lambda i: (0, i, 0)),
            pl.BlockSpec((NC, EB, D), lambda i: (0, i, 0)),
        ],
        out_specs=pl.BlockSpec((NC, EB, W), lambda i: (0, i, 0)),
        out_shape=jax.ShapeDtypeStruct((NC, E_PAD, W), jnp.float32),
    )(qd_all, kv_all)


def _scatter_body(tidx_hbm, sv_hbm, zrow_hbm, out_hbm,
                  tidx_v, ob, acc, sem):
    c = lax.axis_index("c")
    s = lax.axis_index("s")

    for r in range(NR):
        pltpu.sync_copy(zrow_hbm, acc.at[pl.ds(s * ROWS, ROWS)])
        plsc.subcore_barrier()

        def chunk_body(t, carry):
            base = s * EPT + t * CH
            pltpu.sync_copy(tidx_hbm.at[r, pl.ds(base, CH)], tidx_v)
            pltpu.sync_copy(sv_hbm.at[c, pl.ds(base, CH)], ob)
            pltpu.sync_copy(ob, acc.at[tidx_v], add=True)
            return carry

        lax.fori_loop(0, NCHUNK, chunk_body, 0)
        plsc.subcore_barrier()
        pltpu.sync_copy(acc.at[pl.ds(s * ROWS, ROWS)],
                        out_hbm.at[c, r, pl.ds(s * ROWS, ROWS)])
        plsc.subcore_barrier()


_scatter = pl.kernel(
    _scatter_body,
    out_type=jax.ShapeDtypeStruct((NC, NR, HROWS, W), jnp.float32),
    mesh=plsc.VectorSubcoreMesh(core_axis_name="c", subcore_axis_name="s",
                                num_cores=NC, num_subcores=NS),
    scratch_types=[
        pltpu.VMEM((CH,), jnp.int32),
        pltpu.VMEM((CH, W), jnp.float32),
        pltpu.VMEM_SHARED((HROWS, W), jnp.float32),
        pltpu.SemaphoreType.DMA,
    ],
)


def _final_body(uu_ref, ui_ref, zu_ref, zi_ref):
    for u_ref, z_ref in ((uu_ref, zu_ref), (ui_ref, zi_ref)):
        u = u_ref[...]
        parts = []
        for cc in range(NC):
            for j in range(2):
                num = u[cc, 0, :, DH * j:DH * (j + 1)]
                den = jnp.maximum(u[cc, 0, :, 2 * DH + j:2 * DH + j + 1],
                                  1e-30)
                parts.append(num / den)
        z_ref[...] = jnp.maximum(jnp.concatenate(parts, axis=1), 0.0)


def _final(u_user, u_item):
    nblk = HALF // FBLK
    return pl.pallas_call(
        _final_body,
        grid=(NR * nblk,),
        in_specs=[
            pl.BlockSpec((NC, 1, FBLK, W),
                         lambda i: (0, i // nblk, i % nblk, 0)),
            pl.BlockSpec((NC, 1, FBLK, W),
                         lambda i: (0, i // nblk, i % nblk, 0)),
        ],
        out_specs=[
            pl.BlockSpec((FBLK, D), lambda i: (i, 0)),
            pl.BlockSpec((FBLK, D), lambda i: (i, 0)),
        ],
        out_shape=[
            jax.ShapeDtypeStruct((NR * HALF, D), jnp.float32),
            jax.ShapeDtypeStruct((NR * HALF, D), jnp.float32),
        ],
    )(u_user, u_item)


def _wcat(w):
    return jnp.transpose(w, (1, 0, 2)).reshape(D, H * DH)


def _prep_w(wq, wk, wv):
    Wq = _wcat(wq)
    Wk = _wcat(wk)
    Wv = _wcat(wv)
    wkv2 = jnp.stack([
        jnp.concatenate([Wk[:, :2 * DH], Wv[:, :2 * DH]], axis=1),
        jnp.concatenate([Wk[:, 2 * DH:], Wv[:, 2 * DH:]], axis=1),
    ])
    return Wq, wkv2


def _prep_edges(ei):
    dst = jnp.concatenate(
        [ei[0].astype(jnp.int32), jnp.full((E_PAD - E,), N, jnp.int32)])
    src = jnp.concatenate(
        [ei[1].astype(jnp.int32), jnp.zeros((E_PAD - E,), jnp.int32)])
    src2 = jnp.stack([src + cc * N_PAD for cc in range(NC)])
    d = dst[None, :] - jnp.arange(NR, dtype=jnp.int32)[:, None] * HALF
    tidx = jnp.where((d >= 0) & (d < HALF), d, HALF)
    return dst, src2, tidx


def _attend(dst, src2, tidx, q_t, kv2_t, zrow):
    qd_all, kv_all = _gather(dst, src2, q_t, kv2_t)
    sv = _scale(qd_all, kv_all)
    return _scatter(tidx, sv, zrow)


def kernel(u2i_edge_index, i2u_edge_index, h_user, h_item,
           user_w_q, user_w_k, user_w_v, item_w_q, item_w_k, item_w_v):
    hu = jnp.pad(h_user, ((0, N_PAD - N), (0, 0)))
    hi = jnp.pad(h_item, ((0, N_PAD - N), (0, 0)))
    uwq, uwkv2 = _prep_w(user_w_q, user_w_k, user_w_v)
    iwq, iwkv2 = _prep_w(item_w_q, item_w_k, item_w_v)
    q_u, kv2_u = _proj(hu, uwq, uwkv2)
    q_i, kv2_i = _proj(hi, iwq, iwkv2)

    zrow = jnp.zeros((ROWS, W), jnp.float32)
    du, su2, tu = _prep_edges(i2u_edge_index)
    di, si2, ti = _prep_edges(u2i_edge_index)
    u_user = _attend(du, su2, tu, q_u, kv2_i, zrow)
    u_item = _attend(di, si2, ti, q_i, kv2_u, zrow)

    z_user, z_item = _final(u_user, u_item)
    return (z_user[:N], z_item[:N])
